# baseline (speedup 1.0000x reference)
import jax
import jax.numpy as jnp
from jax.experimental import pallas as pl
from jax.experimental.pallas import tpu as pltpu
from jax.experimental.pallas import tpu_sc as plsc


_GATHER_WINDOW = 128


def _sc_gather_pairs(table_pairs, idx_phys):
    batch = idx_phys.shape[0]
    width = table_pairs.shape[1]
    idx = idx_phys.reshape(1, batch)
    mesh = plsc.VectorSubcoreMesh(core_axis_name="core",
                                  subcore_axis_name="subcore")

    @pl.kernel(
        out_type=jax.ShapeDtypeStruct((batch, width), table_pairs.dtype),
        mesh=mesh,
    )
    def gather_kernel(table_hbm, idx_hbm, out_hbm):
        def body(idx_vmem, out_vmem):
            pltpu.sync_copy(table_hbm.at[idx_vmem.at[0]], out_vmem)

        pltpu.emit_pipeline(
            body,
            grid=(batch // _GATHER_WINDOW,),
            in_specs=[pl.BlockSpec((1, _GATHER_WINDOW),
                                   index_map=lambda i: (0, i))],
            out_specs=[pl.BlockSpec((_GATHER_WINDOW, width),
                                    index_map=lambda i: (i, 0))],
            core_axis_name=("core", "subcore"),
            dimension_semantics=(pltpu.PARALLEL,),
        )(idx_hbm, out_hbm)

    return gather_kernel(table_pairs, idx)


_VOCAB_TILE = 2048


def _tc_matmul(embeds, W):
    batch, embed = embeds.shape
    vocab = W.shape[0]

    def mm_kernel(a_ref, w_ref, o_ref):
        a = a_ref[...].astype(jnp.bfloat16)
        w = w_ref[...].astype(jnp.bfloat16)
        o_ref[...] = jax.lax.dot_general(
            a, w, (((1,), (1,)), ((), ())),
            preferred_element_type=jnp.float32)

    return pl.pallas_call(
        mm_kernel,
        grid=(pl.cdiv(vocab, _VOCAB_TILE),),
        in_specs=[
            pl.BlockSpec((batch, embed), lambda i: (0, 0)),
            pl.BlockSpec((_VOCAB_TILE, embed), lambda i: (i, 0)),
        ],
        out_specs=pl.BlockSpec((batch, _VOCAB_TILE), lambda i: (0, i)),
        out_shape=jax.ShapeDtypeStruct((batch, vocab), jnp.float32),
    )(embeds, W)


def kernel(x, emb_table, W):
    vocab, embed = emb_table.shape
    table_pairs = emb_table.reshape(vocab // 2, 2 * embed)
    pairs = _sc_gather_pairs(table_pairs, (x >> 1).astype(jnp.int32))
    odd = (x & 1).astype(jnp.bool_).reshape(-1, 1)
    embeds = jnp.where(odd, pairs[:, embed:], pairs[:, :embed])
    return _tc_matmul(embeds, W)


# speedup vs baseline: 2.8003x; 2.8003x over previous
import jax
import jax.numpy as jnp
from jax.experimental import pallas as pl
from jax.experimental.pallas import tpu as pltpu
from jax.experimental.pallas import tpu_sc as plsc


_GATHER_WINDOW = 128


def _sc_gather_pairs(table_pairs, idx_phys):
    batch = idx_phys.shape[0]
    width = table_pairs.shape[1]
    idx = idx_phys.reshape(1, batch)
    mesh = plsc.VectorSubcoreMesh(core_axis_name="core",
                                  subcore_axis_name="subcore")

    @pl.kernel(
        out_type=jax.ShapeDtypeStruct((batch, width), table_pairs.dtype),
        mesh=mesh,
    )
    def gather_kernel(table_hbm, idx_hbm, out_hbm):
        def body(idx_vmem, out_vmem):
            pltpu.sync_copy(table_hbm.at[idx_vmem.at[0]], out_vmem)

        pltpu.emit_pipeline(
            body,
            grid=(batch // _GATHER_WINDOW,),
            in_specs=[pl.BlockSpec((1, _GATHER_WINDOW),
                                   index_map=lambda i: (0, i))],
            out_specs=[pl.BlockSpec((_GATHER_WINDOW, width),
                                    index_map=lambda i: (i, 0))],
            core_axis_name=("core", "subcore"),
            dimension_semantics=(pltpu.PARALLEL,),
        )(idx_hbm, out_hbm)

    return gather_kernel(table_pairs, idx)


_VOCAB_TILE = 2048


def _tc_matmul_t(Wt, at):
    embed, vocab = Wt.shape
    batch = at.shape[1]

    def mm_kernel(w_ref, a_ref, o_ref):
        w = w_ref[...].astype(jnp.bfloat16)
        a = a_ref[...].astype(jnp.bfloat16)
        o_ref[...] = jax.lax.dot_general(
            w, a, (((0,), (0,)), ((), ())),
            preferred_element_type=jnp.float32)

    return pl.pallas_call(
        mm_kernel,
        grid=(pl.cdiv(vocab, _VOCAB_TILE),),
        in_specs=[
            pl.BlockSpec((embed, _VOCAB_TILE), lambda i: (0, i)),
            pl.BlockSpec((embed, batch), lambda i: (0, 0)),
        ],
        out_specs=pl.BlockSpec((_VOCAB_TILE, batch), lambda i: (i, 0)),
        out_shape=jax.ShapeDtypeStruct((vocab, batch), jnp.float32),
    )(Wt, at)


def kernel(x, emb_table, W):
    vocab, embed = emb_table.shape
    table_pairs = emb_table.reshape(vocab // 2, 2 * embed)
    pairs = _sc_gather_pairs(table_pairs, (x >> 1).astype(jnp.int32))
    odd = (x & 1).astype(jnp.bool_).reshape(-1, 1)
    at = jnp.where(odd, pairs[:, embed:], pairs[:, :embed]).T
    outT = _tc_matmul_t(W.T, at)
    return outT.T
